# baseline (speedup 1.0000x reference)
import jax
import jax.numpy as jnp
from jax import lax
from jax.experimental import pallas as pl
from jax.experimental.pallas import tpu as pltpu
from jax.experimental.pallas import tpu_sc as plsc

_N = 10000
_D = 128
_T = 4
_EPT = 80000
_NC = 2
_NS = 16
_NW = _NC * _NS
_E = _T * _EPT
_EW = _E // _NW
_C = 125
_EWP = 10000
_J = _EWP // _C
_NH = 2
_JH = _J // _NH
_NTR = 16
_RC = 80
_NCH = _N // _RC
_KR = -(-_NCH // _NS)
_NPAD = 10240
_IW = _NPAD // _NW
_IC = 80
_IJ = _IW // _IC

_MESH = plsc.VectorSubcoreMesh(
    core_axis_name="c", subcore_axis_name="s", num_cores=_NC, num_subcores=_NS)


def _embed_body(table, idx, out, idxv, rows, sem):
    c = lax.axis_index("c")
    s = lax.axis_index("s")
    wid = c * _NS + s
    pltpu.sync_copy(idx.at[wid], idxv)
    for k in range(_IJ):
        pltpu.async_copy(table.at[idxv.at[k]], rows, sem).wait()
        pltpu.sync_copy(rows, out.at[pl.ds(wid * _IW + k * _IC, _IC)])


_embed = pl.kernel(
    _embed_body,
    out_type=jax.ShapeDtypeStruct((_NPAD, _D), jnp.float32),
    mesh=_MESH,
    scratch_types=[
        pltpu.VMEM((_IJ, _IC), jnp.int32),
        pltpu.VMEM((_IC, _D), jnp.float32),
        pltpu.SemaphoreType.DMA,
    ],
)


def _agg_body(y, srcg, tgtg, ztile, out, srcv, tgtv, rows_a, rows_b, accs,
              sem_a, sem_b, sem_z):
    c = lax.axis_index("c")
    s = lax.axis_index("s")
    wid = c * _NS + s
    for k in range(_KR):
        ch = s + _NS * k

        @pl.when(ch < _NCH)
        def _():
            pltpu.async_copy(ztile, accs.at[pl.ds(ch * _RC, _RC)], sem_z)

    pltpu.sync_copy(srcg.at[wid, 0], srcv)
    pltpu.sync_copy(tgtg.at[wid, 0], tgtv)
    pltpu.async_copy(y.at[srcv.at[0]], rows_a, sem_a)
    for k in range(_KR):
        ch = s + _NS * k

        @pl.when(ch < _NCH)
        def _():
            pltpu.make_async_copy(ztile, accs.at[pl.ds(ch * _RC, _RC)], sem_z).wait()

    plsc.subcore_barrier()

    for half in range(_NH):
        if half > 0:
            pltpu.sync_copy(srcg.at[wid, half], srcv)
            pltpu.sync_copy(tgtg.at[wid, half], tgtv)
            pltpu.async_copy(y.at[srcv.at[0]], rows_a, sem_a)

        def body(i, carry):
            j = 2 * i
            pltpu.async_copy(y.at[srcv.at[j + 1]], rows_b, sem_b)
            pltpu.make_async_copy(y.at[srcv.at[j]], rows_a, sem_a).wait()
            pltpu.sync_copy(rows_a, accs.at[tgtv.at[j]], add=True)

            @pl.when(j + 2 < _JH)
            def _():
                pltpu.async_copy(y.at[srcv.at[j + 2]], rows_a, sem_a)

            pltpu.make_async_copy(y.at[srcv.at[j + 1]], rows_b, sem_b).wait()
            pltpu.sync_copy(rows_b, accs.at[tgtv.at[j + 1]], add=True)
            return carry

        lax.fori_loop(0, _JH // 2, body, 0)
    plsc.subcore_barrier()
    for k in range(_KR):
        ch = s + _NS * k

        @pl.when(ch < _NCH)
        def _():
            pltpu.async_copy(
                accs.at[pl.ds(ch * _RC, _RC)], out.at[c, pl.ds(ch * _RC, _RC)], sem_z)

    for k in range(_KR):
        ch = s + _NS * k

        @pl.when(ch < _NCH)
        def _():
            pltpu.make_async_copy(
                accs.at[pl.ds(ch * _RC, _RC)], out.at[c, pl.ds(ch * _RC, _RC)], sem_z).wait()


_agg = pl.kernel(
    _agg_body,
    out_type=jax.ShapeDtypeStruct((_NC, _N, _D), jnp.float32),
    mesh=_MESH,
    scratch_types=[
        pltpu.VMEM((_JH, _C), jnp.int32),
        pltpu.VMEM((_JH, _C), jnp.int32),
        pltpu.VMEM((_C, _D), jnp.float32),
        pltpu.VMEM((_C, _D), jnp.float32),
        pltpu.VMEM_SHARED((_N + _NTR, _D), jnp.float32),
        pltpu.SemaphoreType.DMA,
        pltpu.SemaphoreType.DMA,
        pltpu.SemaphoreType.DMA,
    ],
)

_BN = 400


def _y0_body(h_ref, w_ref, b_ref, y_ref):
    y_ref[0] = jnp.dot(h_ref[...], w_ref[0], preferred_element_type=jnp.float32) + b_ref[0]


_y0 = pl.pallas_call(
    _y0_body,
    grid=(_T, _N // _BN),
    in_specs=[
        pl.BlockSpec((_BN, _D), lambda t, i: (i, 0)),
        pl.BlockSpec((1, _D, _D), lambda t, i: (t, 0, 0)),
        pl.BlockSpec((1, 1, _D), lambda t, i: (t, 0, 0)),
    ],
    out_specs=pl.BlockSpec((1, _BN, _D), lambda t, i: (t, i, 0)),
    out_shape=jax.ShapeDtypeStruct((_T, _N, _D), jnp.float32),
)


def _fused_body(p_ref, h_ref, wx_ref, wh_ref, gb_ref, wn_ref, bn_ref, hn_ref, y_ref):
    h = h_ref[...]
    agg = p_ref[0] + p_ref[1]
    xg = jnp.dot(agg, wx_ref[...], preferred_element_type=jnp.float32) + gb_ref[0]
    hg = jnp.dot(h, wh_ref[...], preferred_element_type=jnp.float32)
    z = jax.nn.sigmoid(xg[:, :_D] + hg[:, :_D])
    r = jax.nn.sigmoid(xg[:, _D:2 * _D] + hg[:, _D:2 * _D])
    hh = jnp.tanh(xg[:, 2 * _D:] + r * hg[:, 2 * _D:])
    hn = z * h + (1.0 - z) * hh
    hn_ref[...] = hn
    for t in range(_T):
        y_ref[t] = jnp.dot(hn, wn_ref[t], preferred_element_type=jnp.float32) + bn_ref[t]


_fused = pl.pallas_call(
    _fused_body,
    grid=(_N // _BN,),
    in_specs=[
        pl.BlockSpec((2, _BN, _D), lambda i: (0, i, 0)),
        pl.BlockSpec((_BN, _D), lambda i: (i, 0)),
        pl.BlockSpec((_D, 3 * _D), lambda i: (0, 0)),
        pl.BlockSpec((_D, 3 * _D), lambda i: (0, 0)),
        pl.BlockSpec((1, 3 * _D), lambda i: (0, 0)),
        pl.BlockSpec((_T, _D, _D), lambda i: (0, 0, 0)),
        pl.BlockSpec((_T, 1, _D), lambda i: (0, 0, 0)),
    ],
    out_specs=[
        pl.BlockSpec((_BN, _D), lambda i: (i, 0)),
        pl.BlockSpec((_T, _BN, _D), lambda i: (0, i, 0)),
    ],
    out_shape=[
        jax.ShapeDtypeStruct((_N, _D), jnp.float32),
        jax.ShapeDtypeStruct((_T, _N, _D), jnp.float32),
    ],
)


def kernel(node_ids, node_locs, edge_index, embedding, type_W, type_b, gru_Wx, gru_Wh, gru_b):
    del node_locs
    ids = node_ids.astype(jnp.int32)
    ids_pad = jnp.concatenate(
        [ids, jnp.zeros((_NPAD - _N,), jnp.int32)]).reshape(_NW, _IJ, _IC)
    ei = edge_index.astype(jnp.int32)
    npad = _EWP - _EW
    src = (ei[:, 0, :] + (jnp.arange(_T, dtype=jnp.int32) * _N)[:, None]).reshape(_NW, _EW)
    src = jnp.concatenate(
        [src, jnp.zeros((_NW, npad), jnp.int32)], axis=1).reshape(_NW, _NH, _JH, _C)
    tgt = ei[:, 1, :].reshape(_NW, _EW)
    pad_tgt = jnp.broadcast_to(
        _N + (jnp.arange(npad, dtype=jnp.int32) % _NTR), (_NW, npad))
    tgt = jnp.concatenate([tgt, pad_tgt], axis=1).reshape(_NW, _NH, _JH, _C)
    ztile = jnp.zeros((_RC, _D), jnp.float32)
    gb2 = gru_b.reshape(2, 1, 3 * _D)

    tb3 = type_b.reshape(2, _T, 1, _D)
    h = _embed(embedding, ids_pad)[:_N]
    y = _y0(h, type_W[0], tb3[0])
    step_layer = (0, 0, 0, 1)
    next_layer = (0, 0, 1, 1)
    for stp in range(4):
        l, nl = step_layer[stp], next_layer[stp]
        p = _agg(y.reshape(_T * _N, _D), src, tgt, ztile)
        h, y = _fused(p, h, gru_Wx[l], gru_Wh[l], gb2[l], type_W[nl], tb3[nl])
    return h


# speedup vs baseline: 1.0677x; 1.0677x over previous
import jax
import jax.numpy as jnp
from jax import lax
from jax.experimental import pallas as pl
from jax.experimental.pallas import tpu as pltpu
from jax.experimental.pallas import tpu_sc as plsc

_N = 10000
_D = 128
_T = 4
_EPT = 80000
_NC = 2
_NS = 16
_NW = _NC * _NS
_E = _T * _EPT
_EW = _E // _NW
_C = 125
_EWP = 10000
_J = _EWP // _C
_NH = 2
_JH = _J // _NH
_NTR = 16
_RC = 80
_NCH = _N // _RC
_KR = -(-_NCH // _NS)
_NPAD = 10240
_IW = _NPAD // _NW
_IC = 80
_IJ = _IW // _IC

_MESH = plsc.VectorSubcoreMesh(
    core_axis_name="c", subcore_axis_name="s", num_cores=_NC, num_subcores=_NS)


def _embed_body(table, idx, out, idxv, rows, sem):
    c = lax.axis_index("c")
    s = lax.axis_index("s")
    wid = c * _NS + s
    pltpu.sync_copy(idx.at[wid], idxv)
    for k in range(_IJ):
        pltpu.async_copy(table.at[idxv.at[k]], rows, sem).wait()
        pltpu.sync_copy(rows, out.at[pl.ds(wid * _IW + k * _IC, _IC)])


_embed = pl.kernel(
    _embed_body,
    out_type=jax.ShapeDtypeStruct((_NPAD, _D), jnp.float32),
    mesh=_MESH,
    scratch_types=[
        pltpu.VMEM((_IJ, _IC), jnp.int32),
        pltpu.VMEM((_IC, _D), jnp.float32),
        pltpu.SemaphoreType.DMA,
    ],
)


def _agg_body(y, srcg, tgtg, ztile, out, srcv, tgtv, rows_a, rows_b, accs,
              sem_a, sem_b):
    c = lax.axis_index("c")
    s = lax.axis_index("s")
    wid = c * _NS + s
    rbuf = rows_a.at[pl.ds(0, _RC)]
    pltpu.sync_copy(ztile, rbuf)
    for k in range(_KR):
        ch = s + _NS * k

        @pl.when(ch < _NCH)
        def _():
            pltpu.sync_copy(rbuf, accs.at[pl.ds(ch * _RC, _RC)])

    plsc.subcore_barrier()

    for half in range(_NH):
        pltpu.sync_copy(srcg.at[wid, half], srcv)
        pltpu.sync_copy(tgtg.at[wid, half], tgtv)
        pltpu.async_copy(y.at[srcv.at[0]], rows_a, sem_a)

        def body(i, carry):
            j = 2 * i
            pltpu.async_copy(y.at[srcv.at[j + 1]], rows_b, sem_b)
            pltpu.make_async_copy(y.at[srcv.at[j]], rows_a, sem_a).wait()
            pltpu.sync_copy(rows_a, accs.at[tgtv.at[j]], add=True)

            @pl.when(j + 2 < _JH)
            def _():
                pltpu.async_copy(y.at[srcv.at[j + 2]], rows_a, sem_a)

            pltpu.make_async_copy(y.at[srcv.at[j + 1]], rows_b, sem_b).wait()
            pltpu.sync_copy(rows_b, accs.at[tgtv.at[j + 1]], add=True)
            return carry

        lax.fori_loop(0, _JH // 2, body, 0)
    plsc.subcore_barrier()
    for k in range(_KR):
        ch = s + _NS * k

        @pl.when(ch < _NCH)
        def _():
            pltpu.sync_copy(accs.at[pl.ds(ch * _RC, _RC)], out.at[c, pl.ds(ch * _RC, _RC)])


_agg = pl.kernel(
    _agg_body,
    out_type=jax.ShapeDtypeStruct((_NC, _N, _D), jnp.float32),
    mesh=_MESH,
    scratch_types=[
        pltpu.VMEM((_JH, _C), jnp.int32),
        pltpu.VMEM((_JH, _C), jnp.int32),
        pltpu.VMEM((_C, _D), jnp.float32),
        pltpu.VMEM((_C, _D), jnp.float32),
        pltpu.VMEM_SHARED((_N + _NTR, _D), jnp.float32),
        pltpu.SemaphoreType.DMA,
        pltpu.SemaphoreType.DMA,
    ],
)

_BN = 400


def _y0_body(h_ref, w_ref, b_ref, y_ref):
    y_ref[0] = jnp.dot(h_ref[...], w_ref[0], preferred_element_type=jnp.float32) + b_ref[0]


_y0 = pl.pallas_call(
    _y0_body,
    grid=(_T, _N // _BN),
    in_specs=[
        pl.BlockSpec((_BN, _D), lambda t, i: (i, 0)),
        pl.BlockSpec((1, _D, _D), lambda t, i: (t, 0, 0)),
        pl.BlockSpec((1, 1, _D), lambda t, i: (t, 0, 0)),
    ],
    out_specs=pl.BlockSpec((1, _BN, _D), lambda t, i: (t, i, 0)),
    out_shape=jax.ShapeDtypeStruct((_T, _N, _D), jnp.float32),
)


def _fused_body(p_ref, h_ref, wx_ref, wh_ref, gb_ref, wn_ref, bn_ref, hn_ref, y_ref):
    h = h_ref[...]
    agg = p_ref[0] + p_ref[1]
    xg = jnp.dot(agg, wx_ref[...], preferred_element_type=jnp.float32) + gb_ref[0]
    hg = jnp.dot(h, wh_ref[...], preferred_element_type=jnp.float32)
    z = jax.nn.sigmoid(xg[:, :_D] + hg[:, :_D])
    r = jax.nn.sigmoid(xg[:, _D:2 * _D] + hg[:, _D:2 * _D])
    hh = jnp.tanh(xg[:, 2 * _D:] + r * hg[:, 2 * _D:])
    hn = z * h + (1.0 - z) * hh
    hn_ref[...] = hn
    for t in range(_T):
        y_ref[t] = jnp.dot(hn, wn_ref[t], preferred_element_type=jnp.float32) + bn_ref[t]


_fused = pl.pallas_call(
    _fused_body,
    grid=(_N // _BN,),
    in_specs=[
        pl.BlockSpec((2, _BN, _D), lambda i: (0, i, 0)),
        pl.BlockSpec((_BN, _D), lambda i: (i, 0)),
        pl.BlockSpec((_D, 3 * _D), lambda i: (0, 0)),
        pl.BlockSpec((_D, 3 * _D), lambda i: (0, 0)),
        pl.BlockSpec((1, 3 * _D), lambda i: (0, 0)),
        pl.BlockSpec((_T, _D, _D), lambda i: (0, 0, 0)),
        pl.BlockSpec((_T, 1, _D), lambda i: (0, 0, 0)),
    ],
    out_specs=[
        pl.BlockSpec((_BN, _D), lambda i: (i, 0)),
        pl.BlockSpec((_T, _BN, _D), lambda i: (0, i, 0)),
    ],
    out_shape=[
        jax.ShapeDtypeStruct((_N, _D), jnp.float32),
        jax.ShapeDtypeStruct((_T, _N, _D), jnp.float32),
    ],
)


def kernel(node_ids, node_locs, edge_index, embedding, type_W, type_b, gru_Wx, gru_Wh, gru_b):
    del node_locs
    ids = node_ids.astype(jnp.int32)
    ids_pad = jnp.concatenate(
        [ids, jnp.zeros((_NPAD - _N,), jnp.int32)]).reshape(_NW, _IJ, _IC)
    ei = edge_index.astype(jnp.int32)
    npad = _EWP - _EW
    src = (ei[:, 0, :] + (jnp.arange(_T, dtype=jnp.int32) * _N)[:, None]).reshape(_NW, _EW)
    src = jnp.concatenate(
        [src, jnp.zeros((_NW, npad), jnp.int32)], axis=1).reshape(_NW, _NH, _JH, _C)
    tgt = ei[:, 1, :].reshape(_NW, _EW)
    pad_tgt = jnp.broadcast_to(
        _N + (jnp.arange(npad, dtype=jnp.int32) % _NTR), (_NW, npad))
    tgt = jnp.concatenate([tgt, pad_tgt], axis=1).reshape(_NW, _NH, _JH, _C)
    ztile = jnp.zeros((_RC, _D), jnp.float32)
    gb2 = gru_b.reshape(2, 1, 3 * _D)

    tb3 = type_b.reshape(2, _T, 1, _D)
    h = _embed(embedding, ids_pad)[:_N]
    y = _y0(h, type_W[0], tb3[0])
    step_layer = (0, 0, 0, 1)
    next_layer = (0, 0, 1, 1)
    for stp in range(4):
        l, nl = step_layer[stp], next_layer[stp]
        p = _agg(y.reshape(_T * _N, _D), src, tgt, ztile)
        h, y = _fused(p, h, gru_Wx[l], gru_Wh[l], gb2[l], type_W[nl], tb3[nl])
    return h


# speedup vs baseline: 1.2276x; 1.1497x over previous
import jax
import jax.numpy as jnp
from jax import lax
from jax.experimental import pallas as pl
from jax.experimental.pallas import tpu as pltpu
from jax.experimental.pallas import tpu_sc as plsc

_N = 10000
_D = 128
_T = 4
_EPT = 80000
_NC = 2
_NS = 16
_NW = _NC * _NS
_E = _T * _EPT
_EW = _E // _NW
_C = 125
_EWP = 10000
_J = _EWP // _C
_NH = 2
_JH = _J // _NH
_NTR = 16
_RC = 80
_NCH = _N // _RC
_KR = -(-_NCH // _NS)
_NPAD = 10240
_IW = _NPAD // _NW
_IC = 80
_IJ = _IW // _IC

_MESH = plsc.VectorSubcoreMesh(
    core_axis_name="c", subcore_axis_name="s", num_cores=_NC, num_subcores=_NS)


def _embed_body(table, idx, out, idxv, rows, sem):
    c = lax.axis_index("c")
    s = lax.axis_index("s")
    wid = c * _NS + s
    pltpu.sync_copy(idx.at[wid], idxv)
    for k in range(_IJ):
        pltpu.async_copy(table.at[idxv.at[k]], rows, sem).wait()
        pltpu.sync_copy(rows, out.at[pl.ds(wid * _IW + k * _IC, _IC)])


_embed = pl.kernel(
    _embed_body,
    out_type=jax.ShapeDtypeStruct((_NPAD, _D), jnp.float32),
    mesh=_MESH,
    scratch_types=[
        pltpu.VMEM((_IJ, _IC), jnp.int32),
        pltpu.VMEM((_IC, _D), jnp.float32),
        pltpu.SemaphoreType.DMA,
    ],
)


def _agg_body(y, srcg, tgtg, ztile, out, srcv, tgtv, rows_a, rows_b, accs,
              sem_a, sem_b):
    c = lax.axis_index("c")
    s = lax.axis_index("s")
    wid = c * _NS + s
    rbuf = rows_a.at[pl.ds(0, _RC)]
    pltpu.sync_copy(ztile, rbuf)
    for k in range(_KR):
        ch = s + _NS * k

        @pl.when(ch < _NCH)
        def _():
            pltpu.sync_copy(rbuf, accs.at[pl.ds(ch * _RC, _RC)])

    plsc.subcore_barrier()

    for half in range(_NH):
        pltpu.sync_copy(srcg.at[wid, half], srcv)
        pltpu.sync_copy(tgtg.at[wid, half], tgtv)
        pltpu.async_copy(y.at[srcv.at[0]], rows_a, sem_a)

        def body(i, carry):
            j = 2 * i
            pltpu.async_copy(y.at[srcv.at[j + 1]], rows_b, sem_b)
            pltpu.make_async_copy(y.at[srcv.at[j]], rows_a, sem_a).wait()
            pltpu.sync_copy(rows_a, accs.at[tgtv.at[j]], add=True)

            @pl.when(j + 2 < _JH)
            def _():
                pltpu.async_copy(y.at[srcv.at[j + 2]], rows_a, sem_a)

            pltpu.make_async_copy(y.at[srcv.at[j + 1]], rows_b, sem_b).wait()
            pltpu.sync_copy(rows_b, accs.at[tgtv.at[j + 1]], add=True)
            return carry

        lax.fori_loop(0, _JH // 2, body, 0)
    plsc.subcore_barrier()
    for k in range(_KR):
        ch = s + _NS * k

        @pl.when(ch < _NCH)
        def _():
            pltpu.sync_copy(accs.at[pl.ds(ch * _RC, _RC)], out.at[c, pl.ds(ch * _RC, _RC)])


_agg = pl.kernel(
    _agg_body,
    out_type=jax.ShapeDtypeStruct((_NC, _N, _D), jnp.float32),
    mesh=_MESH,
    scratch_types=[
        pltpu.VMEM((_JH, _C), jnp.int32),
        pltpu.VMEM((_JH, _C), jnp.int32),
        pltpu.VMEM((_C, _D), jnp.float32),
        pltpu.VMEM((_C, _D), jnp.float32),
        pltpu.VMEM_SHARED((_N + _NTR, _D), jnp.float32),
        pltpu.SemaphoreType.DMA,
        pltpu.SemaphoreType.DMA,
    ],
)

_BN = 2000


def _y0_body(h_ref, w_ref, b_ref, y_ref):
    y_ref[0] = jnp.dot(h_ref[...], w_ref[0], preferred_element_type=jnp.float32) + b_ref[0]


_y0 = pl.pallas_call(
    _y0_body,
    grid=(_T, _N // _BN),
    in_specs=[
        pl.BlockSpec((_BN, _D), lambda t, i: (i, 0)),
        pl.BlockSpec((1, _D, _D), lambda t, i: (t, 0, 0)),
        pl.BlockSpec((1, 1, _D), lambda t, i: (t, 0, 0)),
    ],
    out_specs=pl.BlockSpec((1, _BN, _D), lambda t, i: (t, i, 0)),
    out_shape=jax.ShapeDtypeStruct((_T, _N, _D), jnp.float32),
)


def _fused_body(p_ref, h_ref, wx_ref, wh_ref, gb_ref, wn_ref, bn_ref, hn_ref, y_ref):
    h = h_ref[...]
    agg = p_ref[0] + p_ref[1]
    xg = jnp.dot(agg, wx_ref[...], preferred_element_type=jnp.float32) + gb_ref[0]
    hg = jnp.dot(h, wh_ref[...], preferred_element_type=jnp.float32)
    z = jax.nn.sigmoid(xg[:, :_D] + hg[:, :_D])
    r = jax.nn.sigmoid(xg[:, _D:2 * _D] + hg[:, _D:2 * _D])
    hh = jnp.tanh(xg[:, 2 * _D:] + r * hg[:, 2 * _D:])
    hn = z * h + (1.0 - z) * hh
    hn_ref[...] = hn
    for t in range(_T):
        y_ref[t] = jnp.dot(hn, wn_ref[t], preferred_element_type=jnp.float32) + bn_ref[t]


_fused = pl.pallas_call(
    _fused_body,
    grid=(_N // _BN,),
    in_specs=[
        pl.BlockSpec((2, _BN, _D), lambda i: (0, i, 0)),
        pl.BlockSpec((_BN, _D), lambda i: (i, 0)),
        pl.BlockSpec((_D, 3 * _D), lambda i: (0, 0)),
        pl.BlockSpec((_D, 3 * _D), lambda i: (0, 0)),
        pl.BlockSpec((1, 3 * _D), lambda i: (0, 0)),
        pl.BlockSpec((_T, _D, _D), lambda i: (0, 0, 0)),
        pl.BlockSpec((_T, 1, _D), lambda i: (0, 0, 0)),
    ],
    out_specs=[
        pl.BlockSpec((_BN, _D), lambda i: (i, 0)),
        pl.BlockSpec((_T, _BN, _D), lambda i: (0, i, 0)),
    ],
    out_shape=[
        jax.ShapeDtypeStruct((_N, _D), jnp.float32),
        jax.ShapeDtypeStruct((_T, _N, _D), jnp.float32),
    ],
)


def kernel(node_ids, node_locs, edge_index, embedding, type_W, type_b, gru_Wx, gru_Wh, gru_b):
    del node_locs
    ids = node_ids.astype(jnp.int32)
    ids_pad = jnp.concatenate(
        [ids, jnp.zeros((_NPAD - _N,), jnp.int32)]).reshape(_NW, _IJ, _IC)
    ei = edge_index.astype(jnp.int32)
    npad = _EWP - _EW
    src = (ei[:, 0, :] + (jnp.arange(_T, dtype=jnp.int32) * _N)[:, None]).reshape(_NW, _EW)
    src = jnp.concatenate(
        [src, jnp.zeros((_NW, npad), jnp.int32)], axis=1).reshape(_NW, _NH, _JH, _C)
    tgt = ei[:, 1, :].reshape(_NW, _EW)
    pad_tgt = jnp.broadcast_to(
        _N + (jnp.arange(npad, dtype=jnp.int32) % _NTR), (_NW, npad))
    tgt = jnp.concatenate([tgt, pad_tgt], axis=1).reshape(_NW, _NH, _JH, _C)
    ztile = jnp.zeros((_RC, _D), jnp.float32)
    gb2 = gru_b.reshape(2, 1, 3 * _D)

    tb3 = type_b.reshape(2, _T, 1, _D)
    h = _embed(embedding, ids_pad)[:_N]
    y = _y0(h, type_W[0], tb3[0])
    step_layer = (0, 0, 0, 1)
    next_layer = (0, 0, 1, 1)
    for stp in range(4):
        l, nl = step_layer[stp], next_layer[stp]
        p = _agg(y.reshape(_T * _N, _D), src, tgt, ztile)
        h, y = _fused(p, h, gru_Wx[l], gru_Wh[l], gb2[l], type_W[nl], tb3[nl])
    return h
